# baseline (speedup 1.0000x reference)
import jax
import jax.numpy as jnp
from jax import lax
from jax.experimental import pallas as pl
from jax.experimental.pallas import tpu as pltpu
from jax.experimental.pallas import tpu_sc as plsc

N = 16384
NN = 2048
TPR = NN // 128
NC, NS, L = 2, 16, 16
NW = NC * NS
N_SC = 2048
N_TC = N - N_SC
RPW = N_SC // NW
RB = 8
NIT = RPW // RB
KPB = 128 // L
PTB = (RPW + 127) // 128 + 1
BIG = 2 ** 30


def _sc_body(nodes_hbm, point_hbm, out_hbm, buf0, buf1, pt_v, out_v,
             sem0, sem1):
    wid = lax.axis_index("s") * NC + lax.axis_index("c")
    base_row = wid * RPW
    blk0 = base_row // 128

    iota = lax.iota(jnp.int32, L)
    inf_v = jnp.full((L,), jnp.inf, dtype=jnp.float32)
    zero_i = jnp.zeros((L,), dtype=jnp.int32)
    podd = iota & 1
    poison_blk = lax.shift_right_logical(iota, 1) * TPR + podd * (TPR - 1)
    poison_off = podd * 127
    lane0 = iota == 0

    bufs = (buf0, buf1)
    sems = (sem0, sem1)

    pltpu.sync_copy(point_hbm.at[pl.ds(blk0, PTB)], pt_v)
    pltpu.async_copy(nodes_hbm.at[pl.ds(base_row * TPR, RB * TPR)], buf0, sem0)

    def process_row(buf, r_in_buf, buf_iter):
        lr = buf_iter * RB + r_in_buf
        gr = base_row + lr
        pb = jnp.broadcast_to(lax.shift_right_logical(gr, 7) - blk0, (L,))
        pe = jnp.broadcast_to(gr & 127, (L,))
        px = plsc.load_gather(pt_v, [pb, zero_i, pe])
        py = plsc.load_gather(pt_v, [pb, zero_i + 1, pe])

        def tile(t, carry):
            mv0, mi0, mv1, mi1, nb = carry
            blk = r_in_buf * TPR + t
            for k in range(KPB):
                x = buf[blk, 0, pl.ds(k * L, L)]
                y = buf[blk, 1, pl.ds(k * L, L)]
                dx = x - px
                dy = y - py
                d = dx * dx + dy * dy
                if k % 2 == 0:
                    upd = d < mv0
                    mv0 = jnp.where(upd, d, mv0)
                    mi0 = jnp.where(upd, nb + k * L if k else nb, mi0)
                else:
                    upd = d < mv1
                    mv1 = jnp.where(upd, d, mv1)
                    mi1 = jnp.where(upd, nb + k * L, mi1)
            return mv0, mi0, mv1, mi1, nb + 128

        mv0, mi0, mv1, mi1, _ = lax.fori_loop(
            0, TPR, tile, (inf_v, zero_i, inf_v, zero_i, iota), unroll=2)

        m = jnp.min(jnp.minimum(mv0, mv1))
        best = jnp.min(jnp.minimum(jnp.where(mv0 == m, mi0, BIG),
                                   jnp.where(mv1 == m, mi1, BIG)))
        plsc.store_scatter(out_v, [jnp.broadcast_to(lr, (L,))],
                           jnp.broadcast_to(best, (L,)), mask=lane0)

    def outer(g, carry):
        for b in range(2):
            i = 2 * g + b
            nxt = jnp.minimum(i + 1, NIT - 1)
            pltpu.async_copy(
                nodes_hbm.at[pl.ds((base_row + nxt * RB) * TPR, RB * TPR)],
                bufs[1 - b], sems[1 - b])
            pltpu.make_async_copy(
                nodes_hbm.at[pl.ds(0, RB * TPR)], bufs[b], sems[b]).wait()
            plsc.store_scatter(bufs[b], [poison_blk, zero_i, poison_off],
                               inf_v)
            for r in range(RB):
                process_row(bufs[b], r, i)
        return carry

    lax.fori_loop(0, NIT // 2, outer, 0)
    pltpu.make_async_copy(
        nodes_hbm.at[pl.ds(0, RB * TPR)], bufs[0], sems[0]).wait()
    pltpu.sync_copy(out_v, out_hbm.at[pl.ds(base_row, RPW)])


RPS = 256


def _tc_body(nodes_ref, point_ref, out_ref):
    x = nodes_ref[:, 0, :].reshape(RPS, NN)
    y = nodes_ref[:, 1, :].reshape(RPS, NN)
    px = point_ref[:, 0:1]
    py = point_ref[:, 1:2]
    dx = x - px
    dy = y - py
    d = dx * dx + dy * dy
    nid = lax.broadcasted_iota(jnp.int32, (RPS, NN), 1)
    d = jnp.where((nid >= 1) & (nid <= NN - 2), d, jnp.inf)
    m = jnp.min(d, axis=1, keepdims=True)
    bid = jnp.min(jnp.where(d == m, nid, BIG), axis=1)
    out_ref[:, 0, :] = bid.astype(jnp.int32).reshape(RPS // 128, 128)


@jax.jit
def _run(line_nodes, point):
    nodes_v = (line_nodes.reshape(N, TPR, 128, 2)
               .transpose(0, 1, 3, 2).reshape(N * TPR, 2, 128))
    point_v = point.reshape(N // 128, 128, 2).transpose(0, 2, 1)

    mesh = plsc.VectorSubcoreMesh(
        core_axis_name="c", subcore_axis_name="s",
        num_cores=NC, num_subcores=NS)
    sc_out = pl.kernel(
        _sc_body,
        out_type=jax.ShapeDtypeStruct((N_SC,), jnp.int32),
        mesh=mesh,
        compiler_params=pltpu.CompilerParams(
            needs_layout_passes=False, use_tc_tiling_on_sc=False),
        scratch_types=[
            pltpu.VMEM((RB * TPR, 2, 128), jnp.float32),
            pltpu.VMEM((RB * TPR, 2, 128), jnp.float32),
            pltpu.VMEM((PTB, 2, 128), jnp.float32),
            pltpu.VMEM((RPW,), jnp.int32),
            pltpu.SemaphoreType.DMA,
            pltpu.SemaphoreType.DMA,
        ],
    )(nodes_v, point_v)

    tc_out = pl.pallas_call(
        _tc_body,
        grid=(N_TC // RPS,),
        in_specs=[
            pl.BlockSpec((RPS * TPR, 2, 128),
                         lambda g: (N_SC // RPS + g, 0, 0)),
            pl.BlockSpec((RPS, 2), lambda g: (N_SC // RPS + g, 0)),
        ],
        out_specs=pl.BlockSpec((RPS // 128, 1, 128), lambda g: (g, 0, 0)),
        out_shape=jax.ShapeDtypeStruct((N_TC // 128, 1, 128), jnp.int32),
    )(nodes_v, point)

    return jnp.concatenate([sc_out, tc_out.reshape(N_TC)])


def kernel(line_nodes, point):
    return _run(line_nodes, point)


# speedup vs baseline: 1.5769x; 1.5769x over previous
import jax
import jax.numpy as jnp
from jax import lax
from jax.experimental import pallas as pl
from jax.experimental.pallas import tpu as pltpu
from jax.experimental.pallas import tpu_sc as plsc

N = 16384
NN = 2048
TPR = NN // 128
NC, NS, L = 2, 16, 16
NW = NC * NS
N_SC = 9216
N_TC = N - N_SC
RPW = N_SC // NW
RB = 8
NIT = RPW // RB
KPB = 128 // L
PTB = (RPW + 127) // 128 + 1
BIG = 2 ** 30


def _sc_body(nodes_hbm, point_hbm, out_hbm, buf0, buf1, pt_v, out_v,
             sem0, sem1):
    wid = lax.axis_index("s") * NC + lax.axis_index("c")
    base_row = wid * RPW
    blk0 = base_row // 128

    iota = lax.iota(jnp.int32, L)
    inf_v = jnp.full((L,), jnp.inf, dtype=jnp.float32)
    zero_i = jnp.zeros((L,), dtype=jnp.int32)
    podd = iota & 1
    poison_blk = lax.shift_right_logical(iota, 1) * TPR + podd * (TPR - 1)
    poison_off = podd * 127
    lane0 = iota == 0

    bufs = (buf0, buf1)
    sems = (sem0, sem1)

    pltpu.sync_copy(point_hbm.at[pl.ds(blk0, PTB)], pt_v)
    pltpu.async_copy(nodes_hbm.at[pl.ds(base_row * TPR, RB * TPR)], buf0, sem0)

    def process_row(buf, r_in_buf, buf_iter):
        lr = buf_iter * RB + r_in_buf
        gr = base_row + lr
        pb = jnp.broadcast_to(lax.shift_right_logical(gr, 7) - blk0, (L,))
        pe = jnp.broadcast_to(gr & 127, (L,))
        px = plsc.load_gather(pt_v, [pb, zero_i, pe])
        py = plsc.load_gather(pt_v, [pb, zero_i + 1, pe])

        def tile(t, carry):
            mv0, mi0, mv1, mi1, nb = carry
            blk = r_in_buf * TPR + t
            for k in range(KPB):
                x = buf[blk, 0, pl.ds(k * L, L)]
                y = buf[blk, 1, pl.ds(k * L, L)]
                dx = x - px
                dy = y - py
                d = dx * dx + dy * dy
                if k % 2 == 0:
                    upd = d < mv0
                    mv0 = jnp.where(upd, d, mv0)
                    mi0 = jnp.where(upd, nb + k * L if k else nb, mi0)
                else:
                    upd = d < mv1
                    mv1 = jnp.where(upd, d, mv1)
                    mi1 = jnp.where(upd, nb + k * L, mi1)
            return mv0, mi0, mv1, mi1, nb + 128

        mv0, mi0, mv1, mi1, _ = lax.fori_loop(
            0, TPR, tile, (inf_v, zero_i, inf_v, zero_i, iota), unroll=2)

        m = jnp.min(jnp.minimum(mv0, mv1))
        best = jnp.min(jnp.minimum(jnp.where(mv0 == m, mi0, BIG),
                                   jnp.where(mv1 == m, mi1, BIG)))
        plsc.store_scatter(out_v, [jnp.broadcast_to(lr, (L,))],
                           jnp.broadcast_to(best, (L,)), mask=lane0)

    def outer(g, carry):
        for b in range(2):
            i = 2 * g + b
            nxt = jnp.minimum(i + 1, NIT - 1)
            pltpu.async_copy(
                nodes_hbm.at[pl.ds((base_row + nxt * RB) * TPR, RB * TPR)],
                bufs[1 - b], sems[1 - b])
            pltpu.make_async_copy(
                nodes_hbm.at[pl.ds(0, RB * TPR)], bufs[b], sems[b]).wait()
            plsc.store_scatter(bufs[b], [poison_blk, zero_i, poison_off],
                               inf_v)
            for r in range(RB):
                process_row(bufs[b], r, i)
        return carry

    lax.fori_loop(0, NIT // 2, outer, 0)
    pltpu.make_async_copy(
        nodes_hbm.at[pl.ds(0, RB * TPR)], bufs[0], sems[0]).wait()
    pltpu.sync_copy(out_v, out_hbm.at[pl.ds(base_row, RPW)])


RPS = 256


def _tc_body(nodes_ref, point_ref, out_ref):
    x = nodes_ref[:, 0, :].reshape(RPS, NN)
    y = nodes_ref[:, 1, :].reshape(RPS, NN)
    px = point_ref[:, 0:1]
    py = point_ref[:, 1:2]
    dx = x - px
    dy = y - py
    d = dx * dx + dy * dy
    nid = lax.broadcasted_iota(jnp.int32, (RPS, NN), 1)
    d = jnp.where((nid >= 1) & (nid <= NN - 2), d, jnp.inf)
    m = jnp.min(d, axis=1, keepdims=True)
    bid = jnp.min(jnp.where(d == m, nid, BIG), axis=1)
    out_ref[:, 0, :] = bid.astype(jnp.int32).reshape(RPS // 128, 128)


@jax.jit
def _run(line_nodes, point):
    nodes_v = (line_nodes.reshape(N, TPR, 128, 2)
               .transpose(0, 1, 3, 2).reshape(N * TPR, 2, 128))
    point_v = point.reshape(N // 128, 128, 2).transpose(0, 2, 1)

    mesh = plsc.VectorSubcoreMesh(
        core_axis_name="c", subcore_axis_name="s",
        num_cores=NC, num_subcores=NS)
    sc_out = pl.kernel(
        _sc_body,
        out_type=jax.ShapeDtypeStruct((N_SC,), jnp.int32),
        mesh=mesh,
        compiler_params=pltpu.CompilerParams(
            needs_layout_passes=False, use_tc_tiling_on_sc=False),
        scratch_types=[
            pltpu.VMEM((RB * TPR, 2, 128), jnp.float32),
            pltpu.VMEM((RB * TPR, 2, 128), jnp.float32),
            pltpu.VMEM((PTB, 2, 128), jnp.float32),
            pltpu.VMEM((RPW,), jnp.int32),
            pltpu.SemaphoreType.DMA,
            pltpu.SemaphoreType.DMA,
        ],
    )(nodes_v, point_v)

    tc_out = pl.pallas_call(
        _tc_body,
        grid=(N_TC // RPS,),
        in_specs=[
            pl.BlockSpec((RPS * TPR, 2, 128),
                         lambda g: (N_SC // RPS + g, 0, 0)),
            pl.BlockSpec((RPS, 2), lambda g: (N_SC // RPS + g, 0)),
        ],
        out_specs=pl.BlockSpec((RPS // 128, 1, 128), lambda g: (g, 0, 0)),
        out_shape=jax.ShapeDtypeStruct((N_TC // 128, 1, 128), jnp.int32),
    )(nodes_v, point)

    return jnp.concatenate([sc_out, tc_out.reshape(N_TC)])


def kernel(line_nodes, point):
    return _run(line_nodes, point)
